# baseline (speedup 1.0000x reference)
import functools

import jax
import jax.numpy as jnp
from jax import lax
from jax.experimental import pallas as pl
from jax.experimental.pallas import tpu as pltpu
from jax.experimental.pallas import tpu_sc as plsc

T, D, E, K = 2048, 768, 64, 8
H, BN, O = 1024, 256, 768
CAP = 512
BT = 256
MC = 512
NW = 32
TPW = T // NW


def _router_body(x_ref, wg_ref, b_ref, topw_ref, topi_ref):
    xb = x_ref[...]
    wg = wg_ref[...]
    s = jax.nn.sigmoid(jnp.dot(xb, wg, preferred_element_type=jnp.float32))
    sc = s + b_ref[...]
    iota = lax.broadcasted_iota(jnp.int32, (BT, E), 1)
    vals, idxs = [], []
    for _ in range(K):
        m = jnp.max(sc, axis=1, keepdims=True)
        idxk = jnp.min(jnp.where(sc == m, iota, E), axis=1, keepdims=True)
        oh = iota == idxk
        sc = jnp.where(oh, -jnp.inf, sc)
        vals.append(m)
        idxs.append(idxk)
    v = jnp.concatenate(vals, axis=1)
    denom = jnp.sum(v, axis=1, keepdims=True) + 1e-6
    topw_ref[...] = v / denom
    topi_ref[...] = jnp.concatenate(idxs, axis=1)


def _router(xf, Wg, bias):
    return pl.pallas_call(
        _router_body,
        grid=(T // BT,),
        in_specs=[
            pl.BlockSpec((BT, D), lambda i: (i, 0)),
            pl.BlockSpec((D, E), lambda i: (0, 0)),
            pl.BlockSpec((1, E), lambda i: (0, 0)),
        ],
        out_specs=[
            pl.BlockSpec((BT, K), lambda i: (i, 0)),
            pl.BlockSpec((BT, K), lambda i: (i, 0)),
        ],
        out_shape=[
            jax.ShapeDtypeStruct((T, K), jnp.float32),
            jax.ShapeDtypeStruct((T, K), jnp.int32),
        ],
    )(xf, Wg, bias.reshape(1, E))


def _dispatch_body(topi_ref, topw_ref, slotT_ref, wT_ref, cnt_ref):
    i = pl.program_id(0)

    @pl.when(i == 0)
    def _():
        cnt_ref[...] = jnp.zeros_like(cnt_ref)

    ti = topi_ref[...]
    tw = topw_ref[...]
    iota = lax.broadcasted_iota(jnp.int32, (BT, E), 1)
    ohs = []
    oh = jnp.zeros((BT, E), jnp.int32)
    for k in range(K):
        ohk = (iota == ti[:, k:k + 1]).astype(jnp.int32)
        ohs.append(ohk)
        oh = oh + ohk
    incl = oh
    sh = 1
    while sh < BT:
        incl = incl + jnp.pad(incl, ((sh, 0), (0, 0)))[:BT]
        sh *= 2
    carry = cnt_ref[...]
    rk = carry + incl - 1
    slots = []
    for k in range(K):
        sk = (jnp.sum(jnp.where(ohs[k] == 1, rk, 0), axis=1, keepdims=True)
              + ti[:, k:k + 1] * CAP)
        slots.append(sk)
    slot = jnp.concatenate(slots, axis=1)
    slotT_ref[...] = slot.T
    wT_ref[...] = tw.T
    cnt_ref[...] = carry + incl[BT - 1:BT, :]


def _dispatch(topi, topw):
    return pl.pallas_call(
        _dispatch_body,
        grid=(T // BT,),
        in_specs=[
            pl.BlockSpec((BT, K), lambda i: (i, 0)),
            pl.BlockSpec((BT, K), lambda i: (i, 0)),
        ],
        out_specs=[
            pl.BlockSpec((K, BT), lambda i: (0, i)),
            pl.BlockSpec((K, BT), lambda i: (0, i)),
        ],
        out_shape=[
            jax.ShapeDtypeStruct((K, T), jnp.int32),
            jax.ShapeDtypeStruct((K, T), jnp.float32),
        ],
        scratch_shapes=[pltpu.VMEM((1, E), jnp.int32)],
        compiler_params=pltpu.CompilerParams(
            dimension_semantics=("arbitrary",)),
    )(topi, topw)


def _sc_dispatch(xf, slotT, wT):
    mesh = plsc.VectorSubcoreMesh(core_axis_name="c", subcore_axis_name="s")

    @functools.partial(
        pl.kernel, mesh=mesh,
        out_type=[
            jax.ShapeDtypeStruct((E * CAP, D), jnp.float32),
            jax.ShapeDtypeStruct((E * CAP,), jnp.float32),
        ],
        scratch_types=[
            pltpu.VMEM((TPW, D), jnp.float32),
            pltpu.VMEM((K, TPW), jnp.int32),
            pltpu.VMEM((K, TPW), jnp.float32),
            pltpu.SemaphoreType.DMA,
            pltpu.SemaphoreType.DMA,
        ],
    )
    def k(xf_hbm, slotT_hbm, wT_hbm, xg_hbm, wts_hbm,
          rows_v, idx_v, w_v, sem_x, sem_w):
        wid = lax.axis_index("s") * 2 + lax.axis_index("c")
        base = wid * TPW
        pltpu.sync_copy(xf_hbm.at[pl.ds(base, TPW)], rows_v)
        for kk in range(K):
            pltpu.sync_copy(slotT_hbm.at[kk, pl.ds(base, TPW)], idx_v.at[kk])
            pltpu.sync_copy(wT_hbm.at[kk, pl.ds(base, TPW)], w_v.at[kk])
        copies = []
        for kk in range(K):
            copies.append(
                pltpu.async_copy(rows_v, xg_hbm.at[idx_v.at[kk]], sem_x))
            copies.append(
                pltpu.async_copy(w_v.at[kk], wts_hbm.at[idx_v.at[kk]], sem_w))
        for c in copies:
            c.wait()

    return k(xf, slotT, wT)


def _sc_combine(o, slotT):
    mesh = plsc.VectorSubcoreMesh(core_axis_name="c", subcore_axis_name="s")
    CH = 32
    NCHP = TPW // CH
    NCH = K * NCHP

    @functools.partial(
        pl.kernel, mesh=mesh,
        out_type=jax.ShapeDtypeStruct((T, O), jnp.float32),
        scratch_types=[
            pltpu.VMEM((TPW, O), jnp.float32),
            pltpu.VMEM((2, CH, O), jnp.float32),
            pltpu.VMEM((K, TPW), jnp.int32),
            pltpu.SemaphoreType.DMA,
            pltpu.SemaphoreType.DMA,
            pltpu.SemaphoreType.DMA,
        ],
    )
    def k(o_hbm, slotT_hbm, comb_hbm, acc_v, tmp_v, idx_v, sem0, sem1, sema):
        wid = lax.axis_index("s") * 2 + lax.axis_index("c")
        base = wid * TPW
        for kk in range(K):
            pltpu.sync_copy(slotT_hbm.at[kk, pl.ds(base, TPW)], idx_v.at[kk])
        sems = [sem0, sem1]

        acopies = []
        for h in range(NCHP):
            acopies.append(pltpu.async_copy(
                o_hbm.at[idx_v.at[0, pl.ds(h * CH, CH)]],
                acc_v.at[pl.ds(h * CH, CH)], sema))

        def issue(ci):
            kk, h = divmod(ci, NCHP)
            return pltpu.async_copy(
                o_hbm.at[idx_v.at[kk, pl.ds(h * CH, CH)]],
                tmp_v.at[ci % 2], sems[ci % 2])

        cps = {NCHP: issue(NCHP), NCHP + 1: issue(NCHP + 1)}
        for c in acopies:
            c.wait()
        for ci in range(NCHP, NCH):
            h = ci % NCHP
            buf = ci % 2
            cps[ci].wait()

            def body(r, _):
                row = h * CH + r
                for c in range(O // 16):
                    sl = pl.ds(c * 16, 16)
                    plsc.addupdate(acc_v.at[row, sl], tmp_v[buf, r, sl])
                return 0

            lax.fori_loop(0, CH, body, 0)
            if ci + 2 < NCH:
                cps[ci + 2] = issue(ci + 2)
        pltpu.sync_copy(acc_v, comb_hbm.at[pl.ds(base, TPW)])

    return k(o, slotT)


def _gelu(z):
    return 0.5 * z * (1.0 + lax.erf(z * 0.7071067811865476))


def _ffn_body(xg_ref, w1_ref, b1_ref, w2_ref, b2_ref, w3_ref, b3_ref,
              wl_ref, bl_ref, wt_ref, o_ref):
    xb = xg_ref[...]
    h = jnp.dot(xb, w1_ref[0], preferred_element_type=jnp.float32) + b1_ref[0]
    h = _gelu(h)
    h = jnp.dot(h, w2_ref[0], preferred_element_type=jnp.float32) + b2_ref[0]
    h = _gelu(h)
    z = jnp.dot(h, w3_ref[0], preferred_element_type=jnp.float32) + b3_ref[0]
    o = jnp.dot(z, wl_ref[0], preferred_element_type=jnp.float32) + bl_ref[0]
    o_ref[...] = o * wt_ref[...]


def _ffn(xg, W1, b1, W2, b2, W3, b3, Wl, bl, wt):
    nc = CAP // MC
    return pl.pallas_call(
        _ffn_body,
        grid=(E, nc),
        in_specs=[
            pl.BlockSpec((MC, D), lambda e, c: (e * nc + c, 0)),
            pl.BlockSpec((1, D, H), lambda e, c: (e, 0, 0)),
            pl.BlockSpec((1, 1, H), lambda e, c: (e, 0, 0)),
            pl.BlockSpec((1, H, H), lambda e, c: (e, 0, 0)),
            pl.BlockSpec((1, 1, H), lambda e, c: (e, 0, 0)),
            pl.BlockSpec((1, H, BN), lambda e, c: (e, 0, 0)),
            pl.BlockSpec((1, 1, BN), lambda e, c: (e, 0, 0)),
            pl.BlockSpec((1, BN, O), lambda e, c: (e, 0, 0)),
            pl.BlockSpec((1, 1, O), lambda e, c: (e, 0, 0)),
            pl.BlockSpec((MC, 1), lambda e, c: (e * nc + c, 0)),
        ],
        out_specs=pl.BlockSpec((MC, O), lambda e, c: (e * nc + c, 0)),
        out_shape=jax.ShapeDtypeStruct((E * CAP, O), jnp.float32),
        compiler_params=pltpu.CompilerParams(
            dimension_semantics=("arbitrary", "arbitrary"),
            vmem_limit_bytes=110 * 1024 * 1024,
        ),
    )(xg, W1, b1.reshape(E, 1, H), W2, b2.reshape(E, 1, H),
      W3, b3.reshape(E, 1, BN), Wl, bl.reshape(E, 1, O), wt)


def _ln_body(c_ref, g_ref, b_ref, o_ref):
    cb = c_ref[...]
    mu = jnp.mean(cb, axis=1, keepdims=True)
    d = cb - mu
    var = jnp.mean(d * d, axis=1, keepdims=True)
    o_ref[...] = d * lax.rsqrt(var + 1e-5) * g_ref[...] + b_ref[...]


def _layernorm(combined, gamma, beta):
    return pl.pallas_call(
        _ln_body,
        grid=(T // BT,),
        in_specs=[
            pl.BlockSpec((BT, O), lambda i: (i, 0)),
            pl.BlockSpec((1, O), lambda i: (0, 0)),
            pl.BlockSpec((1, O), lambda i: (0, 0)),
        ],
        out_specs=pl.BlockSpec((BT, O), lambda i: (i, 0)),
        out_shape=jax.ShapeDtypeStruct((T, O), jnp.float32),
    )(combined, gamma.reshape(1, O), beta.reshape(1, O))


def kernel(x, Wg, bias, W1, b1, W2, b2, W3, b3, Wl, bl, gamma, beta):
    Bs, Sl, Dm = x.shape
    xf = x.reshape(T, D)
    topw, topi = _router(xf, Wg, bias)
    slotT, wT = _dispatch(topi, topw)
    xg, wts = _sc_dispatch(xf, slotT, wT)
    o = _ffn(xg, W1, b1, W2, b2, W3, b3, Wl, bl, wts.reshape(E * CAP, 1))
    combined = _sc_combine(o, slotT)
    out = _layernorm(combined, gamma, beta)
    return out.reshape(Bs, Sl, O)


# speedup vs baseline: 1.0109x; 1.0109x over previous
import functools

import jax
import jax.numpy as jnp
from jax import lax
from jax.experimental import pallas as pl
from jax.experimental.pallas import tpu as pltpu
from jax.experimental.pallas import tpu_sc as plsc

T, D, E, K = 2048, 768, 64, 8
H, BN, O = 1024, 256, 768
CAP = 512
BT = 256
MC = 512
NW = 32
TPW = T // NW


def _route_dispatch_body(x_ref, wg_ref, b_ref, slotT_ref, wT_ref, cnt_ref):
    i = pl.program_id(0)

    @pl.when(i == 0)
    def _():
        cnt_ref[...] = jnp.zeros_like(cnt_ref)

    xb = x_ref[...]
    wg = wg_ref[...]
    s = jax.nn.sigmoid(jnp.dot(xb, wg, preferred_element_type=jnp.float32))
    sc = s + b_ref[...]
    iota = lax.broadcasted_iota(jnp.int32, (BT, E), 1)
    vals, idxs, ohs = [], [], []
    oh = jnp.zeros((BT, E), jnp.int32)
    for _ in range(K):
        m = jnp.max(sc, axis=1, keepdims=True)
        idxk = jnp.min(jnp.where(sc == m, iota, E), axis=1, keepdims=True)
        ohk = (iota == idxk).astype(jnp.int32)
        sc = jnp.where(ohk == 1, -jnp.inf, sc)
        vals.append(m)
        idxs.append(idxk)
        ohs.append(ohk)
        oh = oh + ohk
    v = jnp.concatenate(vals, axis=1)
    denom = jnp.sum(v, axis=1, keepdims=True) + 1e-6
    tw = v / denom
    incl = oh
    sh = 1
    while sh < BT:
        incl = incl + jnp.pad(incl, ((sh, 0), (0, 0)))[:BT]
        sh *= 2
    carry = cnt_ref[...]
    rk = carry + incl - 1
    slots = []
    for k in range(K):
        sk = (jnp.sum(jnp.where(ohs[k] == 1, rk, 0), axis=1, keepdims=True)
              + idxs[k] * CAP)
        slots.append(sk)
    slot = jnp.concatenate(slots, axis=1)
    slotT_ref[...] = slot.T
    wT_ref[...] = tw.T
    cnt_ref[...] = carry + incl[BT - 1:BT, :]


def _route_dispatch(xf, Wg, bias):
    return pl.pallas_call(
        _route_dispatch_body,
        grid=(T // BT,),
        in_specs=[
            pl.BlockSpec((BT, D), lambda i: (i, 0)),
            pl.BlockSpec((D, E), lambda i: (0, 0)),
            pl.BlockSpec((1, E), lambda i: (0, 0)),
        ],
        out_specs=[
            pl.BlockSpec((K, BT), lambda i: (0, i)),
            pl.BlockSpec((K, BT), lambda i: (0, i)),
        ],
        out_shape=[
            jax.ShapeDtypeStruct((K, T), jnp.int32),
            jax.ShapeDtypeStruct((K, T), jnp.float32),
        ],
        scratch_shapes=[pltpu.VMEM((1, E), jnp.int32)],
        compiler_params=pltpu.CompilerParams(
            dimension_semantics=("arbitrary",)),
    )(xf, Wg, bias.reshape(1, E))


def _sc_dispatch(xf, slotT, wT):
    mesh = plsc.VectorSubcoreMesh(core_axis_name="c", subcore_axis_name="s")

    @functools.partial(
        pl.kernel, mesh=mesh,
        out_type=[
            jax.ShapeDtypeStruct((E * CAP, D), jnp.float32),
            jax.ShapeDtypeStruct((E * CAP,), jnp.float32),
        ],
        scratch_types=[
            pltpu.VMEM((TPW, D), jnp.float32),
            pltpu.VMEM((K, TPW), jnp.int32),
            pltpu.VMEM((K, TPW), jnp.float32),
            pltpu.SemaphoreType.DMA,
            pltpu.SemaphoreType.DMA,
        ],
    )
    def k(xf_hbm, slotT_hbm, wT_hbm, xg_hbm, wts_hbm,
          rows_v, idx_v, w_v, sem_x, sem_w):
        wid = lax.axis_index("s") * 2 + lax.axis_index("c")
        base = wid * TPW
        pltpu.sync_copy(xf_hbm.at[pl.ds(base, TPW)], rows_v)
        for kk in range(K):
            pltpu.sync_copy(slotT_hbm.at[kk, pl.ds(base, TPW)], idx_v.at[kk])
            pltpu.sync_copy(wT_hbm.at[kk, pl.ds(base, TPW)], w_v.at[kk])
        copies = []
        for kk in range(K):
            copies.append(
                pltpu.async_copy(rows_v, xg_hbm.at[idx_v.at[kk]], sem_x))
            copies.append(
                pltpu.async_copy(w_v.at[kk], wts_hbm.at[idx_v.at[kk]], sem_w))
        for c in copies:
            c.wait()

    return k(xf, slotT, wT)


def _sc_combine(o, slotT):
    mesh = plsc.VectorSubcoreMesh(core_axis_name="c", subcore_axis_name="s")
    CH = 32
    NCHP = TPW // CH
    NCH = K * NCHP

    @functools.partial(
        pl.kernel, mesh=mesh,
        out_type=jax.ShapeDtypeStruct((T, O), jnp.float32),
        scratch_types=[
            pltpu.VMEM((TPW, O), jnp.float32),
            pltpu.VMEM((2, CH, O), jnp.float32),
            pltpu.VMEM((K, TPW), jnp.int32),
            pltpu.SemaphoreType.DMA,
            pltpu.SemaphoreType.DMA,
            pltpu.SemaphoreType.DMA,
        ],
    )
    def k(o_hbm, slotT_hbm, comb_hbm, acc_v, tmp_v, idx_v, sem0, sem1, sema):
        wid = lax.axis_index("s") * 2 + lax.axis_index("c")
        base = wid * TPW
        for kk in range(K):
            pltpu.sync_copy(slotT_hbm.at[kk, pl.ds(base, TPW)], idx_v.at[kk])
        sems = [sem0, sem1]

        acopies = []
        for h in range(NCHP):
            acopies.append(pltpu.async_copy(
                o_hbm.at[idx_v.at[0, pl.ds(h * CH, CH)]],
                acc_v.at[pl.ds(h * CH, CH)], sema))

        def issue(ci):
            kk, h = divmod(ci, NCHP)
            return pltpu.async_copy(
                o_hbm.at[idx_v.at[kk, pl.ds(h * CH, CH)]],
                tmp_v.at[ci % 2], sems[ci % 2])

        cps = {NCHP: issue(NCHP), NCHP + 1: issue(NCHP + 1)}
        for c in acopies:
            c.wait()
        for ci in range(NCHP, NCH):
            h = ci % NCHP
            buf = ci % 2
            cps[ci].wait()

            def body(r, _):
                row = h * CH + r
                for c in range(O // 16):
                    sl = pl.ds(c * 16, 16)
                    plsc.addupdate(acc_v.at[row, sl], tmp_v[buf, r, sl])
                return 0

            lax.fori_loop(0, CH, body, 0)
            if ci + 2 < NCH:
                cps[ci + 2] = issue(ci + 2)
        pltpu.sync_copy(acc_v, comb_hbm.at[pl.ds(base, TPW)])

    return k(o, slotT)


def _gelu(z):
    return 0.5 * z * (1.0 + lax.erf(z * 0.7071067811865476))


def _ffn_body(xg_ref, w1_ref, b1_ref, w2_ref, b2_ref, w3_ref, b3_ref,
              wl_ref, bl_ref, wt_ref, o_ref):
    xb = xg_ref[...]
    h = jnp.dot(xb, w1_ref[0], preferred_element_type=jnp.float32) + b1_ref[0]
    h = _gelu(h)
    h = jnp.dot(h, w2_ref[0], preferred_element_type=jnp.float32) + b2_ref[0]
    h = _gelu(h)
    z = jnp.dot(h, w3_ref[0], preferred_element_type=jnp.float32) + b3_ref[0]
    o = jnp.dot(z, wl_ref[0], preferred_element_type=jnp.float32) + bl_ref[0]
    o_ref[...] = o * wt_ref[...]


def _ffn(xg, W1, b1, W2, b2, W3, b3, Wl, bl, wt):
    nc = CAP // MC
    return pl.pallas_call(
        _ffn_body,
        grid=(E, nc),
        in_specs=[
            pl.BlockSpec((MC, D), lambda e, c: (e * nc + c, 0)),
            pl.BlockSpec((1, D, H), lambda e, c: (e, 0, 0)),
            pl.BlockSpec((1, 1, H), lambda e, c: (e, 0, 0)),
            pl.BlockSpec((1, H, H), lambda e, c: (e, 0, 0)),
            pl.BlockSpec((1, 1, H), lambda e, c: (e, 0, 0)),
            pl.BlockSpec((1, H, BN), lambda e, c: (e, 0, 0)),
            pl.BlockSpec((1, 1, BN), lambda e, c: (e, 0, 0)),
            pl.BlockSpec((1, BN, O), lambda e, c: (e, 0, 0)),
            pl.BlockSpec((1, 1, O), lambda e, c: (e, 0, 0)),
            pl.BlockSpec((MC, 1), lambda e, c: (e * nc + c, 0)),
        ],
        out_specs=pl.BlockSpec((MC, O), lambda e, c: (e * nc + c, 0)),
        out_shape=jax.ShapeDtypeStruct((E * CAP, O), jnp.float32),
        compiler_params=pltpu.CompilerParams(
            dimension_semantics=("arbitrary", "arbitrary"),
            vmem_limit_bytes=110 * 1024 * 1024,
        ),
    )(xg, W1, b1.reshape(E, 1, H), W2, b2.reshape(E, 1, H),
      W3, b3.reshape(E, 1, BN), Wl, bl.reshape(E, 1, O), wt)


def _ln_body(c_ref, g_ref, b_ref, o_ref):
    cb = c_ref[...]
    mu = jnp.mean(cb, axis=1, keepdims=True)
    d = cb - mu
    var = jnp.mean(d * d, axis=1, keepdims=True)
    o_ref[...] = d * lax.rsqrt(var + 1e-5) * g_ref[...] + b_ref[...]


def _layernorm(combined, gamma, beta):
    return pl.pallas_call(
        _ln_body,
        grid=(T // BT,),
        in_specs=[
            pl.BlockSpec((BT, O), lambda i: (i, 0)),
            pl.BlockSpec((1, O), lambda i: (0, 0)),
            pl.BlockSpec((1, O), lambda i: (0, 0)),
        ],
        out_specs=pl.BlockSpec((BT, O), lambda i: (i, 0)),
        out_shape=jax.ShapeDtypeStruct((T, O), jnp.float32),
    )(combined, gamma.reshape(1, O), beta.reshape(1, O))


def kernel(x, Wg, bias, W1, b1, W2, b2, W3, b3, Wl, bl, gamma, beta):
    Bs, Sl, Dm = x.shape
    xf = x.reshape(T, D)
    slotT, wT = _route_dispatch(xf, Wg, bias)
    xg, wts = _sc_dispatch(xf, slotT, wT)
    o = _ffn(xg, W1, b1, W2, b2, W3, b3, Wl, bl, wts.reshape(E * CAP, 1))
    combined = _sc_combine(o, slotT)
    out = _layernorm(combined, gamma, beta)
    return out.reshape(Bs, Sl, O)


# speedup vs baseline: 1.0115x; 1.0006x over previous
import functools

import jax
import jax.numpy as jnp
from jax import lax
from jax.experimental import pallas as pl
from jax.experimental.pallas import tpu as pltpu
from jax.experimental.pallas import tpu_sc as plsc

T, D, E, K = 2048, 768, 64, 8
H, BN, O = 1024, 256, 768
CAP = 512
BT = 256
MC = 512
NW = 32
TPW = T // NW


def _route_dispatch_body(x_ref, wg_ref, b_ref, slotT_ref, wT_ref, cnt_ref):
    i = pl.program_id(0)

    @pl.when(i == 0)
    def _():
        cnt_ref[...] = jnp.zeros_like(cnt_ref)

    xb = x_ref[...]
    wg = wg_ref[...]
    s = jax.nn.sigmoid(jnp.dot(xb, wg, preferred_element_type=jnp.float32))
    sc = s + b_ref[...]
    iota = lax.broadcasted_iota(jnp.int32, (BT, E), 1)
    vals, idxs, ohs = [], [], []
    oh = jnp.zeros((BT, E), jnp.int32)
    for _ in range(K):
        m = jnp.max(sc, axis=1, keepdims=True)
        idxk = jnp.min(jnp.where(sc == m, iota, E), axis=1, keepdims=True)
        ohk = (iota == idxk).astype(jnp.int32)
        sc = jnp.where(ohk == 1, -jnp.inf, sc)
        vals.append(m)
        idxs.append(idxk)
        ohs.append(ohk)
        oh = oh + ohk
    v = jnp.concatenate(vals, axis=1)
    denom = jnp.sum(v, axis=1, keepdims=True) + 1e-6
    tw = v / denom
    incl = oh
    sh = 1
    while sh < BT:
        incl = incl + jnp.pad(incl, ((sh, 0), (0, 0)))[:BT]
        sh *= 2
    carry = cnt_ref[...]
    rk = carry + incl - 1
    slots = []
    for k in range(K):
        sk = (jnp.sum(jnp.where(ohs[k] == 1, rk, 0), axis=1, keepdims=True)
              + idxs[k] * CAP)
        slots.append(sk)
    slot = jnp.concatenate(slots, axis=1)
    slotT_ref[...] = slot.T
    wT_ref[...] = tw.T
    cnt_ref[...] = carry + incl[BT - 1:BT, :]


def _route_dispatch(xf, Wg, bias):
    return pl.pallas_call(
        _route_dispatch_body,
        grid=(T // BT,),
        in_specs=[
            pl.BlockSpec((BT, D), lambda i: (i, 0)),
            pl.BlockSpec((D, E), lambda i: (0, 0)),
            pl.BlockSpec((1, E), lambda i: (0, 0)),
        ],
        out_specs=[
            pl.BlockSpec((K, BT), lambda i: (0, i)),
            pl.BlockSpec((K, BT), lambda i: (0, i)),
        ],
        out_shape=[
            jax.ShapeDtypeStruct((K, T), jnp.int32),
            jax.ShapeDtypeStruct((K, T), jnp.float32),
        ],
        scratch_shapes=[pltpu.VMEM((1, E), jnp.int32)],
        compiler_params=pltpu.CompilerParams(
            dimension_semantics=("arbitrary",)),
    )(xf, Wg, bias.reshape(1, E))


def _sc_dispatch(xf, slotT, wT):
    mesh = plsc.VectorSubcoreMesh(core_axis_name="c", subcore_axis_name="s")

    @functools.partial(
        pl.kernel, mesh=mesh,
        out_type=[
            jax.ShapeDtypeStruct((E * CAP, D), jnp.float32),
            jax.ShapeDtypeStruct((E * CAP,), jnp.float32),
        ],
        scratch_types=[
            pltpu.VMEM((TPW, D), jnp.float32),
            pltpu.VMEM((K, TPW), jnp.int32),
            pltpu.VMEM((K, TPW), jnp.float32),
            pltpu.SemaphoreType.DMA,
            pltpu.SemaphoreType.DMA,
        ],
    )
    def k(xf_hbm, slotT_hbm, wT_hbm, xg_hbm, wts_hbm,
          rows_v, idx_v, w_v, sem_x, sem_w):
        wid = lax.axis_index("s") * 2 + lax.axis_index("c")
        base = wid * TPW
        pltpu.sync_copy(xf_hbm.at[pl.ds(base, TPW)], rows_v)
        for kk in range(K):
            pltpu.sync_copy(slotT_hbm.at[kk, pl.ds(base, TPW)], idx_v.at[kk])
            pltpu.sync_copy(wT_hbm.at[kk, pl.ds(base, TPW)], w_v.at[kk])
        copies = []
        for kk in range(K):
            copies.append(
                pltpu.async_copy(rows_v, xg_hbm.at[idx_v.at[kk]], sem_x))
            copies.append(
                pltpu.async_copy(w_v.at[kk], wts_hbm.at[idx_v.at[kk]], sem_w))
        for c in copies:
            c.wait()

    return k(xf, slotT, wT)


def _sc_combine(o, slotT):
    mesh = plsc.VectorSubcoreMesh(core_axis_name="c", subcore_axis_name="s")
    CH = 32
    NCHP = TPW // CH
    NCH = K * NCHP

    @functools.partial(
        pl.kernel, mesh=mesh,
        out_type=jax.ShapeDtypeStruct((T, O), jnp.float32),
        scratch_types=[
            pltpu.VMEM((TPW, O), jnp.float32),
            pltpu.VMEM((2, CH, O), jnp.float32),
            pltpu.VMEM((K, TPW), jnp.int32),
            pltpu.SemaphoreType.DMA,
            pltpu.SemaphoreType.DMA,
            pltpu.SemaphoreType.DMA,
        ],
    )
    def k(o_hbm, slotT_hbm, comb_hbm, acc_v, tmp_v, idx_v, sem0, sem1, sema):
        wid = lax.axis_index("s") * 2 + lax.axis_index("c")
        base = wid * TPW
        for kk in range(K):
            pltpu.sync_copy(slotT_hbm.at[kk, pl.ds(base, TPW)], idx_v.at[kk])
        sems = [sem0, sem1]

        acopies = []
        for h in range(NCHP):
            acopies.append(pltpu.async_copy(
                o_hbm.at[idx_v.at[0, pl.ds(h * CH, CH)]],
                acc_v.at[pl.ds(h * CH, CH)], sema))

        def issue(ci):
            kk, h = divmod(ci, NCHP)
            return pltpu.async_copy(
                o_hbm.at[idx_v.at[kk, pl.ds(h * CH, CH)]],
                tmp_v.at[ci % 2], sems[ci % 2])

        cps = {NCHP: issue(NCHP), NCHP + 1: issue(NCHP + 1)}
        for c in acopies:
            c.wait()
        for ci in range(NCHP, NCH):
            h = ci % NCHP
            buf = ci % 2
            cps[ci].wait()

            def body(r, _):
                row = h * CH + r
                for c in range(O // 16):
                    sl = pl.ds(c * 16, 16)
                    plsc.addupdate(acc_v.at[row, sl], tmp_v[buf, r, sl])
                return 0

            lax.fori_loop(0, CH, body, 0)
            if ci + 2 < NCH:
                cps[ci + 2] = issue(ci + 2)
        pltpu.sync_copy(acc_v, comb_hbm.at[pl.ds(base, TPW)])

    return k(o, slotT)


def _gelu(z):
    return 0.5 * z * (1.0 + lax.erf(z * 0.7071067811865476))


def _ffn_body(xg_ref, w1_ref, b1_ref, w2_ref, b2_ref, w3_ref, b3_ref,
              wl_ref, bl_ref, wt_ref, o_ref):
    bf = jnp.bfloat16
    xb = xg_ref[...].astype(bf)
    h = jnp.dot(xb, w1_ref[0].astype(bf),
                preferred_element_type=jnp.float32) + b1_ref[0]
    h = _gelu(h).astype(bf)
    h = jnp.dot(h, w2_ref[0].astype(bf),
                preferred_element_type=jnp.float32) + b2_ref[0]
    h = _gelu(h).astype(bf)
    z = jnp.dot(h, w3_ref[0].astype(bf),
                preferred_element_type=jnp.float32) + b3_ref[0]
    o = jnp.dot(z.astype(bf), wl_ref[0].astype(bf),
                preferred_element_type=jnp.float32) + bl_ref[0]
    o_ref[...] = o * wt_ref[...]


def _ffn(xg, W1, b1, W2, b2, W3, b3, Wl, bl, wt):
    nc = CAP // MC
    return pl.pallas_call(
        _ffn_body,
        grid=(E, nc),
        in_specs=[
            pl.BlockSpec((MC, D), lambda e, c: (e * nc + c, 0)),
            pl.BlockSpec((1, D, H), lambda e, c: (e, 0, 0)),
            pl.BlockSpec((1, 1, H), lambda e, c: (e, 0, 0)),
            pl.BlockSpec((1, H, H), lambda e, c: (e, 0, 0)),
            pl.BlockSpec((1, 1, H), lambda e, c: (e, 0, 0)),
            pl.BlockSpec((1, H, BN), lambda e, c: (e, 0, 0)),
            pl.BlockSpec((1, 1, BN), lambda e, c: (e, 0, 0)),
            pl.BlockSpec((1, BN, O), lambda e, c: (e, 0, 0)),
            pl.BlockSpec((1, 1, O), lambda e, c: (e, 0, 0)),
            pl.BlockSpec((MC, 1), lambda e, c: (e * nc + c, 0)),
        ],
        out_specs=pl.BlockSpec((MC, O), lambda e, c: (e * nc + c, 0)),
        out_shape=jax.ShapeDtypeStruct((E * CAP, O), jnp.float32),
        compiler_params=pltpu.CompilerParams(
            dimension_semantics=("arbitrary", "arbitrary"),
            vmem_limit_bytes=110 * 1024 * 1024,
        ),
    )(xg, W1, b1.reshape(E, 1, H), W2, b2.reshape(E, 1, H),
      W3, b3.reshape(E, 1, BN), Wl, bl.reshape(E, 1, O), wt)


def _ln_body(c_ref, g_ref, b_ref, o_ref):
    cb = c_ref[...]
    mu = jnp.mean(cb, axis=1, keepdims=True)
    d = cb - mu
    var = jnp.mean(d * d, axis=1, keepdims=True)
    o_ref[...] = d * lax.rsqrt(var + 1e-5) * g_ref[...] + b_ref[...]


def _layernorm(combined, gamma, beta):
    return pl.pallas_call(
        _ln_body,
        grid=(T // BT,),
        in_specs=[
            pl.BlockSpec((BT, O), lambda i: (i, 0)),
            pl.BlockSpec((1, O), lambda i: (0, 0)),
            pl.BlockSpec((1, O), lambda i: (0, 0)),
        ],
        out_specs=pl.BlockSpec((BT, O), lambda i: (i, 0)),
        out_shape=jax.ShapeDtypeStruct((T, O), jnp.float32),
    )(combined, gamma.reshape(1, O), beta.reshape(1, O))


def kernel(x, Wg, bias, W1, b1, W2, b2, W3, b3, Wl, bl, gamma, beta):
    Bs, Sl, Dm = x.shape
    xf = x.reshape(T, D)
    slotT, wT = _route_dispatch(xf, Wg, bias)
    xg, wts = _sc_dispatch(xf, slotT, wT)
    o = _ffn(xg, W1, b1, W2, b2, W3, b3, Wl, bl, wts.reshape(E * CAP, 1))
    combined = _sc_combine(o, slotT)
    out = _layernorm(combined, gamma, beta)
    return out.reshape(Bs, Sl, O)
